# baseline (speedup 1.0000x reference)
import functools

import jax
import jax.numpy as jnp
from jax import lax
from jax.experimental import pallas as pl
from jax.experimental.pallas import tpu as pltpu
from jax.experimental.pallas import tpu_sc as plsc

BATCH = 16384
EMBED_DIM = 64
NUM_WORKERS = 32
B_PER_W = BATCH // NUM_WORKERS


def _dot_kernel(uid_hbm, iid_hbm, uemb_hbm, iemb_hbm, ibias_hbm, out_hbm,
                uidx_v, iidx_v, urows_v, irows_v, bias_v, out_v, sem):
    wid = lax.axis_index("s") * 2 + lax.axis_index("c")
    base = wid * B_PER_W

    pltpu.sync_copy(uid_hbm.at[pl.ds(base, B_PER_W)], uidx_v)
    pltpu.sync_copy(iid_hbm.at[pl.ds(base, B_PER_W)], iidx_v)

    cu = pltpu.async_copy(uemb_hbm.at[uidx_v], urows_v, sem)
    ci = pltpu.async_copy(iemb_hbm.at[iidx_v], irows_v, sem)
    cb = pltpu.async_copy(ibias_hbm.at[iidx_v], bias_v, sem)
    cu.wait()
    ci.wait()
    cb.wait()

    iota16 = lax.iota(jnp.int32, 16)

    def group(g, carry):
        rows = g * 16 + iota16
        accs = [jnp.zeros((16,), jnp.float32) for _ in range(4)]
        for d in range(EMBED_DIM):
            col = jnp.full((16,), d, jnp.int32)
            u = plsc.load_gather(urows_v, [rows, col])
            w = plsc.load_gather(irows_v, [rows, col])
            accs[d % 4] = accs[d % 4] + u * w
        total = (accs[0] + accs[1]) + (accs[2] + accs[3])
        out_v[pl.ds(g * 16, 16)] = total + bias_v[pl.ds(g * 16, 16)]
        return carry

    lax.fori_loop(0, B_PER_W // 16, group, 0)

    pltpu.sync_copy(out_v, out_hbm.at[pl.ds(base, B_PER_W)])


@jax.jit
def kernel(user_ids, item_ids, user_embed, item_embed, item_bias):
    mesh = plsc.VectorSubcoreMesh(core_axis_name="c", subcore_axis_name="s")
    run = functools.partial(
        pl.kernel,
        mesh=mesh,
        compiler_params=pltpu.CompilerParams(
            needs_layout_passes=False, use_tc_tiling_on_sc=False),
        out_type=jax.ShapeDtypeStruct((BATCH,), jnp.float32),
        scratch_types=[
            pltpu.VMEM((B_PER_W,), jnp.int32),
            pltpu.VMEM((B_PER_W,), jnp.int32),
            pltpu.VMEM((B_PER_W, EMBED_DIM), jnp.float32),
            pltpu.VMEM((B_PER_W, EMBED_DIM), jnp.float32),
            pltpu.VMEM((B_PER_W,), jnp.float32),
            pltpu.VMEM((B_PER_W,), jnp.float32),
            pltpu.SemaphoreType.DMA,
        ],
    )(_dot_kernel)
    return run(user_ids.astype(jnp.int32), item_ids.astype(jnp.int32),
               user_embed, item_embed, item_bias.reshape(-1))
